# baseline (speedup 1.0000x reference)
import functools

import jax
import jax.numpy as jnp
from jax.experimental import pallas as pl
from jax.experimental.pallas import tpu as pltpu

_W = 128
_R = 4096


def _decode_threshold(cand):
    pos = cand >= jnp.uint32(0x80000000)
    b = jnp.where(pos, cand & jnp.uint32(0x7FFFFFFF), ~cand)
    f = jax.lax.bitcast_convert_type(b, jnp.float32)
    return jnp.where(cand >= jnp.uint32(0xFF800000), jnp.float32(jnp.inf), f)


def _median_shift_kernel(x_hbm, med_ref, nt_ref, o_hbm,
                         buf, e16, stage, in_sems, out_sem, *, rank):
    j = pl.program_id(0)
    ng = pl.num_programs(0)
    n = buf.shape[1]
    slot = jax.lax.rem(j, 2)

    def in_copy(jj):
        return pltpu.make_async_copy(
            x_hbm.at[:, pl.ds(jj * _W, _W)],
            buf.at[jax.lax.rem(jj, 2)],
            in_sems.at[jax.lax.rem(jj, 2)],
        )

    def out_copy(jj):
        return pltpu.make_async_copy(
            stage, o_hbm.at[:, pl.ds(jj * _W, _W)], out_sem)

    @pl.when(j == 0)
    def _():
        in_copy(j).start()

    @pl.when(j + 1 < ng)
    def _():
        in_copy(j + 1).start()

    in_copy(j).wait()

    kplus1 = jnp.int32(rank + 1)
    nchunks = n // _R

    def enc_body(r, accs):
        rows = pl.ds(r * _R, _R)
        ch = buf[slot, rows, :]
        b32 = jax.lax.bitcast_convert_type(ch, jnp.uint32)
        t = b32 >> jnp.uint32(16)
        neg = b32 >= jnp.uint32(0x80000000)
        enc = jnp.where(neg, ~t & jnp.uint32(0xFFFF), t | jnp.uint32(0x8000))
        e16[rows, :] = (enc ^ jnp.uint32(0x8000)
                        ).astype(jnp.int32).astype(jnp.int16)
        m = (ch <= jnp.float32(-0.0)).reshape(_R // 8, 8, _W)
        accs = list(accs)
        for q in range(_R // 8):
            a = accs[q % 8]
            accs[q % 8] = jnp.where(m[q], a + 1, a)
        return tuple(accs)

    zero8 = jnp.zeros((8, _W), jnp.int32)
    accs0 = jax.lax.fori_loop(0, nchunks, enc_body, tuple([zero8] * 8))
    acc80 = ((accs0[0] + accs0[1]) + (accs0[2] + accs0[3])) + (
        (accs0[4] + accs0[5]) + (accs0[6] + accs0[7]))
    cnt0 = jnp.sum(acc80, axis=0, keepdims=True)
    prefix0 = jnp.where(cnt0 >= kplus1,
                        jnp.zeros((1, _W), jnp.uint32),
                        jnp.full((1, _W), 0x80000000, jnp.uint32))

    def hi_body(i, prefix):
        bit = jnp.uint32(31) - i.astype(jnp.uint32)
        low_mask = (jnp.uint32(1) << bit) - jnp.uint32(1)
        cand = prefix | low_mask
        thr16 = ((cand >> jnp.uint32(16)) ^ jnp.uint32(0x8000)
                 ).astype(jnp.int32).astype(jnp.int16)

        def chunk_body(r, accs):
            ch = e16[pl.ds(r * _R, _R), :]
            m = (ch <= thr16).reshape(_R // 16, 16, _W)
            accs = list(accs)
            for q in range(_R // 16):
                a = accs[q % 8]
                accs[q % 8] = jnp.where(m[q], a + jnp.int16(1), a)
            return tuple(accs)

        zero16 = jnp.zeros((16, _W), jnp.int16)
        accs = jax.lax.fori_loop(0, nchunks, chunk_body, tuple([zero16] * 8))
        acc16 = ((accs[0] + accs[1]) + (accs[2] + accs[3])) + (
            (accs[4] + accs[5]) + (accs[6] + accs[7]))
        cnt = jnp.sum(acc16.astype(jnp.int32), axis=0, keepdims=True)
        return jnp.where(cnt >= kplus1, prefix,
                         prefix | (low_mask + jnp.uint32(1)))

    def bit_body(i, prefix):
        bit = jnp.uint32(31) - i.astype(jnp.uint32)
        low_mask = (jnp.uint32(1) << bit) - jnp.uint32(1)
        cand = prefix | low_mask
        thr = _decode_threshold(cand)

        def chunk_body(r, accs):
            ch = buf[slot, pl.ds(r * _R, _R), :]
            m = (ch <= thr).reshape(_R // 8, 8, _W)
            accs = list(accs)
            for q in range(_R // 8):
                a = accs[q % 8]
                accs[q % 8] = jnp.where(m[q], a + 1, a)
            return tuple(accs)

        zero8 = jnp.zeros((8, _W), jnp.int32)
        accs = jax.lax.fori_loop(
            0, nchunks, chunk_body, tuple([zero8] * 8))
        acc8 = ((accs[0] + accs[1]) + (accs[2] + accs[3])) + ((accs[4] + accs[5]) + (accs[6] + accs[7]))
        cnt = jnp.sum(acc8, axis=0, keepdims=True)
        return jnp.where(cnt >= kplus1, prefix,
                         prefix | (low_mask + jnp.uint32(1)))

    sel = jax.lax.fori_loop(1, 16, hi_body, prefix0)
    sel = jax.lax.fori_loop(16, 32, bit_body, sel)
    med = _decode_threshold(sel)

    nt = nt_ref[0, 0]
    new_med = (med_ref[...] * nt + med) / (nt + jnp.float32(1.0))

    @pl.when(j >= 1)
    def _():
        out_copy(j - 1).wait()

    def sub_body(r, _):
        rows = pl.ds(r * 1024, 1024)
        stage[rows, :] = buf[slot, rows, :] - new_med
        return 0

    jax.lax.fori_loop(0, n // 1024, sub_body, 0)
    out_copy(j).start()

    @pl.when(j == ng - 1)
    def _():
        out_copy(j).wait()


def kernel(x, median, num_track):
    n, c = x.shape
    grid = (c // _W,)
    nt = num_track.astype(jnp.float32).reshape(1, 1)

    fn = functools.partial(_median_shift_kernel, rank=n // 2)
    return pl.pallas_call(
        fn,
        grid=grid,
        in_specs=[
            pl.BlockSpec(memory_space=pltpu.MemorySpace.HBM),
            pl.BlockSpec((1, _W), lambda j: (0, j)),
            pl.BlockSpec(memory_space=pltpu.SMEM),
        ],
        out_specs=pl.BlockSpec(memory_space=pltpu.MemorySpace.HBM),
        out_shape=jax.ShapeDtypeStruct((n, c), jnp.float32),
        scratch_shapes=[
            pltpu.VMEM((2, n, _W), jnp.float32),
            pltpu.VMEM((n, _W), jnp.int16),
            pltpu.VMEM((n, _W), jnp.float32),
            pltpu.SemaphoreType.DMA((2,)),
            pltpu.SemaphoreType.DMA,
        ],
        compiler_params=pltpu.CompilerParams(
            dimension_semantics=("arbitrary",)),
    )(x, median, nt)


# speedup vs baseline: 1.0774x; 1.0774x over previous
import functools

import jax
import jax.numpy as jnp
from jax.experimental import pallas as pl
from jax.experimental.pallas import tpu as pltpu

_W = 128
_R = 4096


def _decode_threshold(cand):
    pos = cand >= jnp.uint32(0x80000000)
    b = jnp.where(pos, cand & jnp.uint32(0x7FFFFFFF), ~cand)
    f = jax.lax.bitcast_convert_type(b, jnp.float32)
    return jnp.where(cand >= jnp.uint32(0xFF800000), jnp.float32(jnp.inf), f)


def _median_shift_kernel(x_hbm, med_ref, nt_ref, o_hbm,
                         buf, stage, in_sems, out_sem, *, rank):
    j = pl.program_id(0)
    ng = pl.num_programs(0)
    n = buf.shape[1]
    slot = jax.lax.rem(j, 2)

    def in_copy(jj):
        return pltpu.make_async_copy(
            x_hbm.at[:, pl.ds(jj * _W, _W)],
            buf.at[jax.lax.rem(jj, 2)],
            in_sems.at[jax.lax.rem(jj, 2)],
        )

    def out_copy(jj):
        return pltpu.make_async_copy(
            stage, o_hbm.at[:, pl.ds(jj * _W, _W)], out_sem)

    @pl.when(j == 0)
    def _():
        in_copy(j).start()

    @pl.when(j + 1 < ng)
    def _():
        in_copy(j + 1).start()

    in_copy(j).wait()

    kplus1 = jnp.int32(rank + 1)
    nchunks = n // _R

    def bit_body(i, prefix):
        bit = jnp.uint32(31) - i.astype(jnp.uint32)
        low_mask = (jnp.uint32(1) << bit) - jnp.uint32(1)
        cand = prefix | low_mask
        thr = _decode_threshold(cand)

        def chunk_body(r, accs):
            ch = buf[slot, pl.ds(r * _R, _R), :]
            m = (ch <= thr).reshape(_R // 8, 8, _W)
            accs = list(accs)
            for q in range(_R // 8):
                a = accs[q % 8]
                accs[q % 8] = jnp.where(m[q], a + 1, a)
            return tuple(accs)

        zero8 = jnp.zeros((8, _W), jnp.int32)
        accs = jax.lax.fori_loop(
            0, nchunks, chunk_body, tuple([zero8] * 8))
        acc8 = ((accs[0] + accs[1]) + (accs[2] + accs[3])) + ((accs[4] + accs[5]) + (accs[6] + accs[7]))
        cnt = jnp.sum(acc8, axis=0, keepdims=True)
        return jnp.where(cnt >= kplus1, prefix,
                         prefix | (low_mask + jnp.uint32(1)))

    prefix0 = jnp.zeros((1, _W), dtype=jnp.uint32)
    sel = jax.lax.fori_loop(0, 32, bit_body, prefix0)
    med = _decode_threshold(sel)

    nt = nt_ref[0, 0]
    new_med = (med_ref[...] * nt + med) / (nt + jnp.float32(1.0))

    @pl.when(j >= 1)
    def _():
        out_copy(j - 1).wait()

    def sub_body(r, _):
        rows = pl.ds(r * 1024, 1024)
        stage[rows, :] = buf[slot, rows, :] - new_med
        return 0

    jax.lax.fori_loop(0, n // 1024, sub_body, 0)
    out_copy(j).start()

    @pl.when(j == ng - 1)
    def _():
        out_copy(j).wait()


def kernel(x, median, num_track):
    n, c = x.shape
    grid = (c // _W,)
    nt = num_track.astype(jnp.float32).reshape(1, 1)

    fn = functools.partial(_median_shift_kernel, rank=n // 2)
    return pl.pallas_call(
        fn,
        grid=grid,
        in_specs=[
            pl.BlockSpec(memory_space=pltpu.MemorySpace.HBM),
            pl.BlockSpec((1, _W), lambda j: (0, j)),
            pl.BlockSpec(memory_space=pltpu.SMEM),
        ],
        out_specs=pl.BlockSpec(memory_space=pltpu.MemorySpace.HBM),
        out_shape=jax.ShapeDtypeStruct((n, c), jnp.float32),
        scratch_shapes=[
            pltpu.VMEM((2, n, _W), jnp.float32),
            pltpu.VMEM((n, _W), jnp.float32),
            pltpu.SemaphoreType.DMA((2,)),
            pltpu.SemaphoreType.DMA,
        ],
        compiler_params=pltpu.CompilerParams(
            dimension_semantics=("arbitrary",)),
    )(x, median, nt)
